# baseline (speedup 1.0000x reference)
import functools
import jax
import jax.numpy as jnp
from jax import lax
from jax.experimental import pallas as pl
from jax.experimental.pallas import tpu as pltpu
from jax.experimental.pallas import tpu_sc as plsc

_B, _N, _F, _T = 8, 1024, 64, 12
_CC, _CT = 64, 64
_E = 16384
_NC, _NS, _L = 2, 16, 16
_RPS = _N // (_NC * _NS)


def _fast_rsqrt(s):
    y = plsc.bitcast(jnp.int32(0x5F3759DF) - (plsc.bitcast(s, jnp.int32) >> 1),
                     jnp.float32)
    for _ in range(3):
        y = y * (1.5 - 0.5 * s * y * y)
    return y


def _lap_sc_body(row_hbm, col_hbm, lap_hbm, er, ec, degp, stage, dinv, acc,
                 deg_sh):
    cid = lax.axis_index("c")
    sid = lax.axis_index("s")
    epw = _E // _NS
    lane = lax.iota(jnp.int32, _L)
    zv = jnp.zeros((_L,), jnp.float32)
    ones = jnp.ones((_L,), jnp.float32)

    pltpu.sync_copy(row_hbm.at[pl.ds(sid * epw, epw)], er.at[pl.ds(0, epw)])
    pltpu.sync_copy(col_hbm.at[pl.ds(sid * epw, epw)], ec.at[pl.ds(0, epw)])

    def _z1(i, _):
        degp[pl.ds(i * _L, _L)] = zv
        return 0
    lax.fori_loop(0, _N // _L, _z1, 0)

    def _p1(i, _):
        r = er[pl.ds(i * _L, _L)]
        c = ec[pl.ds(i * _L, _L)]
        valid = r != c
        for kk in range(_L):
            plsc.addupdate_scatter(degp, [r], ones, mask=valid & (lane == kk))
        return 0
    lax.fori_loop(0, epw // _L, _p1, 0)

    pltpu.sync_copy(degp, deg_sh.at[sid])
    plsc.subcore_barrier()
    pltpu.sync_copy(deg_sh, stage)

    def _m1(j, _):
        s = stage[0, pl.ds(j * _L, _L)]
        for rr in range(1, _NS):
            s = s + stage[rr, pl.ds(j * _L, _L)]
        dinv[pl.ds(j * _L, _L)] = jnp.where(s > 0, _fast_rsqrt(s), 0.0)
        return 0
    lax.fori_loop(0, _N // _L, _m1, 0)

    pltpu.sync_copy(row_hbm, er)
    pltpu.sync_copy(col_hbm, ec)

    def _z2(rr, _):
        def _z2i(j, _2):
            acc[rr, pl.ds(j * _L, _L)] = zv
            return 0
        lax.fori_loop(0, _N // _L, _z2i, 0)
        return 0
    lax.fori_loop(0, _RPS, _z2, 0)

    base = (cid * _NS + sid) * _RPS

    def _p2(i, _):
        r = er[pl.ds(i * _L, _L)]
        c = ec[pl.ds(i * _L, _L)]
        dr = plsc.load_gather(dinv, [r])
        dc = plsc.load_gather(dinv, [c])
        w = -(dr * dc)
        cb = c - base
        inwin = (cb >= 0) & (cb < _RPS) & (r != c)
        for kk in range(_L):
            plsc.addupdate_scatter(acc, [cb, r], w, mask=inwin & (lane == kk))
        return 0
    lax.fori_loop(0, _E // _L, _p2, 0)

    pltpu.sync_copy(acc, lap_hbm.at[pl.ds(base, _RPS)])


def _build_lap(row, col):
    mesh = plsc.VectorSubcoreMesh(core_axis_name="c", subcore_axis_name="s")
    f = functools.partial(
        pl.kernel,
        mesh=mesh,
        out_type=jax.ShapeDtypeStruct((_N, _N), jnp.float32),
        compiler_params=pltpu.CompilerParams(needs_layout_passes=False),
        scratch_types=[
            pltpu.VMEM((_E,), jnp.int32),
            pltpu.VMEM((_E,), jnp.int32),
            pltpu.VMEM((_N,), jnp.float32),
            pltpu.VMEM((_NS, _N), jnp.float32),
            pltpu.VMEM((_N,), jnp.float32),
            pltpu.VMEM((_RPS, _N), jnp.float32),
            pltpu.VMEM_SHARED((_NS, _N), jnp.float32),
        ],
    )(_lap_sc_body)
    return f(row, col)


def _tatt_body(x_ref, u1_ref, u2_ref, u3_ref, be_ref, ve_ref, out_ref,
               lhs_s, rhs_s):
    for t in range(_T):
        xsl = x_ref[0, t]
        v = lax.dot_general(xsl, u1_ref[...], (((1,), (0,)), ((), ())),
                            preferred_element_type=jnp.float32)
        lhs_t = lax.dot_general(v, u2_ref[...], (((0,), (0,)), ((), ())),
                                preferred_element_type=jnp.float32)
        rhs_t = lax.dot_general(u3_ref[...], xsl, (((0,), (0,)), ((), ())),
                                preferred_element_type=jnp.float32)
        lhs_s[t:t + 1, :] = lhs_t
        rhs_s[t:t + 1, :] = rhs_t
    product = lax.dot_general(lhs_s[...], rhs_s[...], (((1,), (1,)), ((), ())),
                              preferred_element_type=jnp.float32)
    e0 = jax.nn.sigmoid(product + be_ref[...])
    emat = jnp.dot(ve_ref[...], e0, preferred_element_type=jnp.float32)
    m = jnp.max(emat, axis=0, keepdims=True)
    ex = jnp.exp(emat - m)
    out_ref[0] = ex / jnp.sum(ex, axis=0, keepdims=True)


def _temporal_att_pallas(xT, u1c, u2, u3c, be2, ve):
    return pl.pallas_call(
        _tatt_body,
        grid=(_B,),
        in_specs=[
            pl.BlockSpec((1, _T, _F, _N), lambda b: (b, 0, 0, 0)),
            pl.BlockSpec((_N, 1), lambda b: (0, 0)),
            pl.BlockSpec((_F, _N), lambda b: (0, 0)),
            pl.BlockSpec((_F, 1), lambda b: (0, 0)),
            pl.BlockSpec((_T, _T), lambda b: (0, 0)),
            pl.BlockSpec((_T, _T), lambda b: (0, 0)),
        ],
        out_specs=pl.BlockSpec((1, _T, _T), lambda b: (b, 0, 0)),
        out_shape=jax.ShapeDtypeStruct((_B, _T, _T), jnp.float32),
        scratch_shapes=[pltpu.VMEM((_T, _N), jnp.float32),
                        pltpu.VMEM((_T, _N), jnp.float32)],
    )(xT, u1c, u2, u3c, be2, ve)


def _xtat_body(xt_ref, x_ref, out_ref):
    out_ref[0] = lax.dot_general(xt_ref[0], x_ref[0], (((0,), (0,)), ((), ())),
                                 preferred_element_type=jnp.float32)


def _apply_tatt(xTf, xtil):
    fn = _F * _N
    nc = 8
    cw = fn // nc
    return pl.pallas_call(
        _xtat_body,
        grid=(_B, nc),
        in_specs=[
            pl.BlockSpec((1, _T, _T), lambda b, c: (b, 0, 0)),
            pl.BlockSpec((1, _T, cw), lambda b, c: (b, 0, c)),
        ],
        out_specs=pl.BlockSpec((1, _T, cw), lambda b, c: (b, 0, c)),
        out_shape=jax.ShapeDtypeStruct((_B, _T, fn), jnp.float32),
    )(xtil, xTf)


def _satt_lr_body(y_ref, w1_ref, w2_ref, w3_ref, lhs_ref, rhs_ref):
    s1 = y_ref[0, 0] * w1_ref[0:1, 0:1]
    for t in range(1, _T):
        s1 += y_ref[0, t] * w1_ref[t:t + 1, 0:1]
    lhs_ref[0] = lax.dot_general(s1, w2_ref[...], (((0,), (0,)), ((), ())),
                                 preferred_element_type=jnp.float32)
    for t in range(_T):
        rhs_ref[0, t:t + 1, :] = lax.dot_general(
            w3_ref[...], y_ref[0, t], (((0,), (0,)), ((), ())),
            preferred_element_type=jnp.float32)


def _satt_lr(y4, w1c, w2, w3c):
    return pl.pallas_call(
        _satt_lr_body,
        grid=(_B,),
        in_specs=[
            pl.BlockSpec((1, _T, _F, _N), lambda b: (b, 0, 0, 0)),
            pl.BlockSpec((_T, 1), lambda b: (0, 0)),
            pl.BlockSpec((_F, _T), lambda b: (0, 0)),
            pl.BlockSpec((_F, 1), lambda b: (0, 0)),
        ],
        out_specs=[
            pl.BlockSpec((1, _N, _T), lambda b: (b, 0, 0)),
            pl.BlockSpec((1, _T, _N), lambda b: (b, 0, 0)),
        ],
        out_shape=[
            jax.ShapeDtypeStruct((_B, _N, _T), jnp.float32),
            jax.ShapeDtypeStruct((_B, _T, _N), jnp.float32),
        ],
    )(y4, w1c, w2, w3c)


def _satt_body(rhs_ref, lhs_ref, bst_ref, vst_ref, out_ref):
    pt = lax.dot_general(rhs_ref[0], lhs_ref[0], (((0,), (1,)), ((), ())),
                         preferred_element_type=jnp.float32)
    sigt = jax.nn.sigmoid(pt + bst_ref[...]).astype(jnp.bfloat16)
    st = jnp.dot(sigt, vst_ref[...], preferred_element_type=jnp.float32)
    m = jnp.max(st, axis=1, keepdims=True)
    ex = jnp.exp(st - m)
    out_ref[0] = (ex / jnp.sum(ex, axis=1, keepdims=True)).astype(jnp.bfloat16)


def _satt(lhs, rhs, bsT, vsT):
    jb = 512
    nj = _N // jb
    return pl.pallas_call(
        _satt_body,
        grid=(_B, nj),
        in_specs=[
            pl.BlockSpec((1, _T, jb), lambda b, j: (b, 0, j)),
            pl.BlockSpec((1, _N, _T), lambda b, j: (b, 0, 0)),
            pl.BlockSpec((jb, _N), lambda b, j: (j, 0)),
            pl.BlockSpec((_N, _N), lambda b, j: (0, 0)),
        ],
        out_specs=pl.BlockSpec((1, jb, _N), lambda b, j: (b, j, 0)),
        out_shape=jax.ShapeDtypeStruct((_B, _N, _N), jnp.bfloat16),
    )(rhs, lhs, bsT, vsT)


def _cheb_body(satt_ref, lap_ref, x_ref, w0_ref, w1_ref, w2_ref, b_ref,
               g_ref, t0_s, t1_s):
    sT = satt_ref[0]
    xb = x_ref[0]
    t0_s[...] = lax.dot_general(sT, xb, (((0,), (0,)), ((), ())),
                                preferred_element_type=jnp.float32)
    m = lap_ref[...] * sT
    t1_s[...] = jnp.dot(m, xb, preferred_element_type=jnp.float32)
    t2 = 2.0 * jnp.dot(m, t1_s[...].astype(jnp.bfloat16),
                       preferred_element_type=jnp.float32) - t0_s[...]
    for t in range(_T):
        sl = slice(t * _F, (t + 1) * _F)
        acc = jnp.dot(t0_s[:, sl], w0_ref[...], preferred_element_type=jnp.float32)
        acc += jnp.dot(t1_s[:, sl], w1_ref[...], preferred_element_type=jnp.float32)
        acc += jnp.dot(t2[:, sl], w2_ref[...], preferred_element_type=jnp.float32)
        acc += b_ref[...]
        g_ref[0, :, t * _CC:(t + 1) * _CC] = jnp.maximum(acc, 0.0).astype(jnp.bfloat16)


def _cheb(sattT, lap, x2, w0, w1, w2, bc):
    return pl.pallas_call(
        _cheb_body,
        grid=(_B,),
        in_specs=[
            pl.BlockSpec((1, _N, _N), lambda b: (b, 0, 0)),
            pl.BlockSpec((_N, _N), lambda b: (0, 0)),
            pl.BlockSpec((1, _N, _T * _F), lambda b: (b, 0, 0)),
            pl.BlockSpec((_F, _CC), lambda b: (0, 0)),
            pl.BlockSpec((_F, _CC), lambda b: (0, 0)),
            pl.BlockSpec((_F, _CC), lambda b: (0, 0)),
            pl.BlockSpec((1, _CC), lambda b: (0, 0)),
        ],
        out_specs=pl.BlockSpec((1, _N, _T * _CC), lambda b: (b, 0, 0)),
        out_shape=jax.ShapeDtypeStruct((_B, _N, _T * _CC), jnp.bfloat16),
        scratch_shapes=[pltpu.VMEM((_N, _T * _F), jnp.float32),
                        pltpu.VMEM((_N, _T * _F), jnp.float32)],
    )(sattT, lap, x2, w0, w1, w2, bc)


def _final_body(g_ref, x_ref, twd_ref, rwt_ref, tb_ref, rb_ref,
                gam_ref, bet_ref, out_ref):
    for t in range(_T):
        acc = jnp.dot(x_ref[0, :, t * _F:(t + 1) * _F], rwt_ref[...],
                      preferred_element_type=jnp.float32) + rb_ref[...]
        for d in range(3):
            tp = t + d - 1
            if 0 <= tp < _T:
                acc += jnp.dot(g_ref[0, :, tp * _CC:(tp + 1) * _CC], twd_ref[d],
                               preferred_element_type=jnp.float32)
        acc += tb_ref[...]
        z = jnp.maximum(acc, 0.0)
        mu = jnp.mean(z, axis=1, keepdims=True)
        dz = z - mu
        var = jnp.mean(dz * dz, axis=1, keepdims=True)
        out_ref[0, t] = dz * lax.rsqrt(var + 1e-5) * gam_ref[...] + bet_ref[...]


def _final(g, x2, twd, rwt, tb1, rb1, gam1, bet1):
    return pl.pallas_call(
        _final_body,
        grid=(_B,),
        in_specs=[
            pl.BlockSpec((1, _N, _T * _CC), lambda b: (b, 0, 0)),
            pl.BlockSpec((1, _N, _T * _F), lambda b: (b, 0, 0)),
            pl.BlockSpec((3, _CC, _CT), lambda b: (0, 0, 0)),
            pl.BlockSpec((_F, _CT), lambda b: (0, 0)),
            pl.BlockSpec((1, _CT), lambda b: (0, 0)),
            pl.BlockSpec((1, _CT), lambda b: (0, 0)),
            pl.BlockSpec((1, _CT), lambda b: (0, 0)),
            pl.BlockSpec((1, _CT), lambda b: (0, 0)),
        ],
        out_specs=pl.BlockSpec((1, _T, _N, _CT), lambda b: (b, 0, 0, 0)),
        out_shape=jax.ShapeDtypeStruct((_B, _T, _N, _CT), jnp.float32),
    )(g, x2, twd, rwt, tb1, rb1, gam1, bet1)


@jax.jit
def _run(x, edge_index, U1, U2, U3, be, Ve, W1, W2, W3, bs, Vs, cheb_w,
         cheb_b, tw, tb, rw, rb, gamma, beta):
    b, n, f, t = x.shape
    xT = jnp.transpose(x, (0, 3, 2, 1))
    x2 = jnp.transpose(x, (0, 1, 3, 2)).reshape(b, n, t * f).astype(jnp.bfloat16)

    lap = _build_lap(edge_index[0], edge_index[1])

    xtil = _temporal_att_pallas(
        xT, U1.reshape(n, 1), U2, U3.reshape(f, 1), be[0], Ve)

    y = _apply_tatt(xT.reshape(b, t, f * n), xtil)
    y4 = y.reshape(b, t, f, n)

    lhs, rhs = _satt_lr(y4, W1.reshape(t, 1), W2, W3.reshape(f, 1))
    sattT = _satt(lhs, rhs, jnp.transpose(bs[0]),
                  jnp.transpose(Vs).astype(jnp.bfloat16))

    g = _cheb(sattT, lap.astype(jnp.bfloat16), x2, cheb_w[0], cheb_w[1],
              cheb_w[2], cheb_b.reshape(1, _CC))

    twd = jnp.transpose(tw[:, :, 0, :], (2, 1, 0)).astype(jnp.bfloat16)
    rwt = jnp.transpose(rw[:, :, 0, 0]).astype(jnp.bfloat16)
    zout = _final(g, x2, twd, rwt, tb.reshape(1, _CT), rb.reshape(1, _CT),
                  gamma.reshape(1, _CT), beta.reshape(1, _CT))
    return jnp.transpose(zout, (0, 2, 3, 1))


def kernel(x, edge_index, U1, U2, U3, be, Ve, W1, W2, W3, bs, Vs, cheb_w,
           cheb_b, tw, tb, rw, rb, gamma, beta):
    return _run(x, edge_index, U1, U2, U3, be, Ve, W1, W2, W3, bs, Vs,
                cheb_w, cheb_b, tw, tb, rw, rb, gamma, beta)


# speedup vs baseline: 1.3545x; 1.3545x over previous
import functools
import jax
import jax.numpy as jnp
from jax import lax
from jax.experimental import pallas as pl
from jax.experimental.pallas import tpu as pltpu
from jax.experimental.pallas import tpu_sc as plsc

_B, _N, _F, _T = 8, 1024, 64, 12
_CC, _CT = 64, 64
_E = 16384
_NC, _NS, _L = 2, 16, 16
_RPS = _N // (_NC * _NS)


def _fast_rsqrt(s):
    y = plsc.bitcast(jnp.int32(0x5F3759DF) - (plsc.bitcast(s, jnp.int32) >> 1),
                     jnp.float32)
    for _ in range(3):
        y = y * (1.5 - 0.5 * s * y * y)
    return y


def _lap_sc_body(row_hbm, col_hbm, lap_hbm, er, ec, degp, stage, dinv, acc,
                 deg_sh):
    cid = lax.axis_index("c")
    sid = lax.axis_index("s")
    epw = _E // _NS
    lane = lax.iota(jnp.int32, _L)
    zv = jnp.zeros((_L,), jnp.float32)
    ones = jnp.ones((_L,), jnp.float32)

    pltpu.sync_copy(row_hbm.at[pl.ds(sid * epw, epw)], er.at[pl.ds(0, epw)])
    pltpu.sync_copy(col_hbm.at[pl.ds(sid * epw, epw)], ec.at[pl.ds(0, epw)])

    def _z1(i, _):
        degp[pl.ds(i * _L, _L)] = zv
        return 0
    lax.fori_loop(0, _N // _L, _z1, 0)

    def _p1(i, _):
        r = er[pl.ds(i * _L, _L)]
        c = ec[pl.ds(i * _L, _L)]
        valid = r != c
        for kk in range(_L):
            plsc.addupdate_scatter(degp, [r], ones, mask=valid & (lane == kk))
        return 0
    lax.fori_loop(0, epw // _L, _p1, 0)

    pltpu.sync_copy(degp, deg_sh.at[sid])
    plsc.subcore_barrier()
    pltpu.sync_copy(deg_sh, stage)

    def _m1(j, _):
        s = stage[0, pl.ds(j * _L, _L)]
        for rr in range(1, _NS):
            s = s + stage[rr, pl.ds(j * _L, _L)]
        dinv[pl.ds(j * _L, _L)] = jnp.where(s > 0, _fast_rsqrt(s), 0.0)
        return 0
    lax.fori_loop(0, _N // _L, _m1, 0)

    pltpu.sync_copy(row_hbm, er)
    pltpu.sync_copy(col_hbm, ec)

    def _z2(rr, _):
        def _z2i(j, _2):
            acc[rr, pl.ds(j * _L, _L)] = zv
            return 0
        lax.fori_loop(0, _N // _L, _z2i, 0)
        return 0
    lax.fori_loop(0, _RPS, _z2, 0)

    base = (cid * _NS + sid) * _RPS

    def _p2(i, _):
        r = er[pl.ds(i * _L, _L)]
        c = ec[pl.ds(i * _L, _L)]
        dr = plsc.load_gather(dinv, [r])
        dc = plsc.load_gather(dinv, [c])
        w = -(dr * dc)
        cb = c - base
        inwin = (cb >= 0) & (cb < _RPS) & (r != c)
        for kk in range(_L):
            plsc.addupdate_scatter(acc, [cb, r], w, mask=inwin & (lane == kk))
        return 0
    lax.fori_loop(0, _E // _L, _p2, 0)

    pltpu.sync_copy(acc, lap_hbm.at[pl.ds(base, _RPS)])


def _build_lap(row, col):
    mesh = plsc.VectorSubcoreMesh(core_axis_name="c", subcore_axis_name="s")
    f = functools.partial(
        pl.kernel,
        mesh=mesh,
        out_type=jax.ShapeDtypeStruct((_N, _N), jnp.float32),
        compiler_params=pltpu.CompilerParams(needs_layout_passes=False),
        scratch_types=[
            pltpu.VMEM((_E,), jnp.int32),
            pltpu.VMEM((_E,), jnp.int32),
            pltpu.VMEM((_N,), jnp.float32),
            pltpu.VMEM((_NS, _N), jnp.float32),
            pltpu.VMEM((_N,), jnp.float32),
            pltpu.VMEM((_RPS, _N), jnp.float32),
            pltpu.VMEM_SHARED((_NS, _N), jnp.float32),
        ],
    )(_lap_sc_body)
    return f(row, col)


def _att_lr_body(x_ref, u1_ref, u2_ref, u3_ref, be_ref, ve_ref, w1_ref,
                 w2_ref, w3_ref, lhs_ref, rhs_ref, lhs_s, rcol_s, q_s):
    for t in range(_T):
        xsl = x_ref[0, :, t * _F:(t + 1) * _F]
        v = lax.dot_general(xsl, u1_ref[...], (((0,), (0,)), ((), ())),
                            preferred_element_type=jnp.float32)
        lhs_s[t:t + 1, :] = lax.dot_general(
            v.astype(jnp.bfloat16), u2_ref[...], (((0,), (0,)), ((), ())),
            preferred_element_type=jnp.float32)
        rcol_s[:, t:t + 1] = lax.dot_general(
            xsl, u3_ref[...], (((1,), (0,)), ((), ())),
            preferred_element_type=jnp.float32)
        q_s[:, t:t + 1] = lax.dot_general(
            xsl, w3_ref[...], (((1,), (0,)), ((), ())),
            preferred_element_type=jnp.float32)
    product = lax.dot_general(lhs_s[...], rcol_s[...], (((1,), (0,)), ((), ())),
                              preferred_element_type=jnp.float32)
    e0 = jax.nn.sigmoid(product + be_ref[...])
    emat = jnp.dot(ve_ref[...], e0, preferred_element_type=jnp.float32)
    m = jnp.max(emat, axis=0, keepdims=True)
    ex = jnp.exp(emat - m)
    xt = ex / jnp.sum(ex, axis=0, keepdims=True)

    a = jnp.dot(xt, w1_ref[...], preferred_element_type=jnp.float32)
    s1 = x_ref[0, :, 0:_F].astype(jnp.float32) * a[0:1, 0:1]
    for t in range(1, _T):
        s1 += x_ref[0, :, t * _F:(t + 1) * _F].astype(jnp.float32) * a[t:t + 1, 0:1]
    lhs_ref[0] = lax.dot_general(s1, w2_ref[...], (((1,), (0,)), ((), ())),
                                 preferred_element_type=jnp.float32)
    rhs_ref[0] = lax.dot_general(xt, q_s[...], (((0,), (1,)), ((), ())),
                                 preferred_element_type=jnp.float32)


def _att_lr(x2, u1c, u2, u3c, be2, ve, w1c, w2, w3c):
    return pl.pallas_call(
        _att_lr_body,
        grid=(_B,),
        in_specs=[
            pl.BlockSpec((1, _N, _T * _F), lambda b: (b, 0, 0)),
            pl.BlockSpec((_N, 1), lambda b: (0, 0)),
            pl.BlockSpec((_F, _N), lambda b: (0, 0)),
            pl.BlockSpec((_F, 1), lambda b: (0, 0)),
            pl.BlockSpec((_T, _T), lambda b: (0, 0)),
            pl.BlockSpec((_T, _T), lambda b: (0, 0)),
            pl.BlockSpec((_T, 1), lambda b: (0, 0)),
            pl.BlockSpec((_F, _T), lambda b: (0, 0)),
            pl.BlockSpec((_F, 1), lambda b: (0, 0)),
        ],
        out_specs=[
            pl.BlockSpec((1, _N, _T), lambda b: (b, 0, 0)),
            pl.BlockSpec((1, _T, _N), lambda b: (b, 0, 0)),
        ],
        out_shape=[
            jax.ShapeDtypeStruct((_B, _N, _T), jnp.float32),
            jax.ShapeDtypeStruct((_B, _T, _N), jnp.float32),
        ],
        scratch_shapes=[pltpu.VMEM((_T, _N), jnp.float32),
                        pltpu.VMEM((_N, _T), jnp.float32),
                        pltpu.VMEM((_N, _T), jnp.float32)],
    )(x2, u1c, u2, u3c, be2, ve, w1c, w2, w3c)


def _satt_body(rhs_ref, lhs_ref, bst_ref, vst_ref, out_ref):
    pt = lax.dot_general(rhs_ref[0], lhs_ref[0], (((0,), (1,)), ((), ())),
                         preferred_element_type=jnp.float32)
    sigt = jax.nn.sigmoid(pt + bst_ref[...]).astype(jnp.bfloat16)
    st = jnp.dot(sigt, vst_ref[...], preferred_element_type=jnp.float32)
    m = jnp.max(st, axis=1, keepdims=True)
    ex = jnp.exp(st - m)
    out_ref[0] = (ex / jnp.sum(ex, axis=1, keepdims=True)).astype(jnp.bfloat16)


def _satt(lhs, rhs, bsT, vsT):
    jb = 512
    nj = _N // jb
    return pl.pallas_call(
        _satt_body,
        grid=(_B, nj),
        in_specs=[
            pl.BlockSpec((1, _T, jb), lambda b, j: (b, 0, j)),
            pl.BlockSpec((1, _N, _T), lambda b, j: (b, 0, 0)),
            pl.BlockSpec((jb, _N), lambda b, j: (j, 0)),
            pl.BlockSpec((_N, _N), lambda b, j: (0, 0)),
        ],
        out_specs=pl.BlockSpec((1, jb, _N), lambda b, j: (b, j, 0)),
        out_shape=jax.ShapeDtypeStruct((_B, _N, _N), jnp.bfloat16),
    )(rhs, lhs, bsT, vsT)


def _cheb_final_body(satt_ref, lap_ref, x_ref, w0_ref, w1_ref, w2_ref, b_ref,
                     twd_ref, rwt_ref, tb_ref, rb_ref, gam_ref, bet_ref,
                     out_ref, t0_s, t1_s, g_s):
    sT = satt_ref[0]
    xb = x_ref[0]
    t0_s[...] = lax.dot_general(sT, xb, (((0,), (0,)), ((), ())),
                                preferred_element_type=jnp.float32)
    m = lap_ref[...] * sT
    t1_s[...] = jnp.dot(m, xb, preferred_element_type=jnp.float32)
    t2 = 2.0 * jnp.dot(m, t1_s[...].astype(jnp.bfloat16),
                       preferred_element_type=jnp.float32) - t0_s[...]
    for t in range(_T):
        sl = slice(t * _F, (t + 1) * _F)
        acc = jnp.dot(t0_s[:, sl], w0_ref[...], preferred_element_type=jnp.float32)
        acc += jnp.dot(t1_s[:, sl], w1_ref[...], preferred_element_type=jnp.float32)
        acc += jnp.dot(t2[:, sl], w2_ref[...], preferred_element_type=jnp.float32)
        acc += b_ref[...]
        g_s[:, t * _CC:(t + 1) * _CC] = jnp.maximum(acc, 0.0).astype(jnp.bfloat16)
    for t in range(_T):
        acc = jnp.dot(x_ref[0, :, t * _F:(t + 1) * _F], rwt_ref[...],
                      preferred_element_type=jnp.float32) + rb_ref[...]
        for d in range(3):
            tp = t + d - 1
            if 0 <= tp < _T:
                acc += jnp.dot(g_s[:, tp * _CC:(tp + 1) * _CC], twd_ref[d],
                               preferred_element_type=jnp.float32)
        acc += tb_ref[...]
        z = jnp.maximum(acc, 0.0)
        mu = jnp.mean(z, axis=1, keepdims=True)
        dz = z - mu
        var = jnp.mean(dz * dz, axis=1, keepdims=True)
        out_ref[0, t] = dz * lax.rsqrt(var + 1e-5) * gam_ref[...] + bet_ref[...]


def _cheb_final(sattT, lap, x2, w0, w1, w2, bc, twd, rwt, tb1, rb1, gam1, bet1):
    return pl.pallas_call(
        _cheb_final_body,
        grid=(_B,),
        in_specs=[
            pl.BlockSpec((1, _N, _N), lambda b: (b, 0, 0)),
            pl.BlockSpec((_N, _N), lambda b: (0, 0)),
            pl.BlockSpec((1, _N, _T * _F), lambda b: (b, 0, 0)),
            pl.BlockSpec((_F, _CC), lambda b: (0, 0)),
            pl.BlockSpec((_F, _CC), lambda b: (0, 0)),
            pl.BlockSpec((_F, _CC), lambda b: (0, 0)),
            pl.BlockSpec((1, _CC), lambda b: (0, 0)),
            pl.BlockSpec((3, _CC, _CT), lambda b: (0, 0, 0)),
            pl.BlockSpec((_F, _CT), lambda b: (0, 0)),
            pl.BlockSpec((1, _CT), lambda b: (0, 0)),
            pl.BlockSpec((1, _CT), lambda b: (0, 0)),
            pl.BlockSpec((1, _CT), lambda b: (0, 0)),
            pl.BlockSpec((1, _CT), lambda b: (0, 0)),
        ],
        out_specs=pl.BlockSpec((1, _T, _N, _CT), lambda b: (b, 0, 0, 0)),
        out_shape=jax.ShapeDtypeStruct((_B, _T, _N, _CT), jnp.float32),
        scratch_shapes=[pltpu.VMEM((_N, _T * _F), jnp.float32),
                        pltpu.VMEM((_N, _T * _F), jnp.float32),
                        pltpu.VMEM((_N, _T * _CC), jnp.bfloat16)],
    )(sattT, lap, x2, w0, w1, w2, bc, twd, rwt, tb1, rb1, gam1, bet1)


@jax.jit
def _run(x, edge_index, U1, U2, U3, be, Ve, W1, W2, W3, bs, Vs, cheb_w,
         cheb_b, tw, tb, rw, rb, gamma, beta):
    b, n, f, t = x.shape
    bf = jnp.bfloat16
    x2 = jnp.transpose(x.astype(bf), (0, 1, 3, 2)).reshape(b, n, t * f)

    lap = _build_lap(edge_index[0], edge_index[1])

    lhs, rhs = _att_lr(x2, U1.reshape(n, 1).astype(bf), U2.astype(bf),
                       U3.reshape(f, 1).astype(bf), be[0], Ve,
                       W1.reshape(t, 1), W2, W3.reshape(f, 1).astype(bf))
    sattT = _satt(lhs, rhs, jnp.transpose(bs[0]),
                  jnp.transpose(Vs).astype(bf))

    twd = jnp.transpose(tw[:, :, 0, :], (2, 1, 0)).astype(bf)
    rwt = jnp.transpose(rw[:, :, 0, 0]).astype(bf)
    zout = _cheb_final(sattT, lap.astype(bf), x2, cheb_w[0], cheb_w[1],
                       cheb_w[2], cheb_b.reshape(1, _CC), twd, rwt,
                       tb.reshape(1, _CT), rb.reshape(1, _CT),
                       gamma.reshape(1, _CT), beta.reshape(1, _CT))
    return jnp.transpose(zout, (0, 2, 3, 1))


def kernel(x, edge_index, U1, U2, U3, be, Ve, W1, W2, W3, bs, Vs, cheb_w,
           cheb_b, tw, tb, rw, rb, gamma, beta):
    return _run(x, edge_index, U1, U2, U3, be, Ve, W1, W2, W3, bs, Vs,
                cheb_w, cheb_b, tw, tb, rw, rb, gamma, beta)


# speedup vs baseline: 1.5370x; 1.1347x over previous
import functools
import jax
import jax.numpy as jnp
from jax import lax
from jax.experimental import pallas as pl
from jax.experimental.pallas import tpu as pltpu
from jax.experimental.pallas import tpu_sc as plsc

_B, _N, _F, _T = 8, 1024, 64, 12
_CC, _CT = 64, 64
_E = 16384
_NC, _NS, _L = 2, 16, 16
_RPS = _N // (_NC * _NS)


def _fast_rsqrt(s):
    y = plsc.bitcast(jnp.int32(0x5F3759DF) - (plsc.bitcast(s, jnp.int32) >> 1),
                     jnp.float32)
    for _ in range(3):
        y = y * (1.5 - 0.5 * s * y * y)
    return y


def _lap_sc_body(row_hbm, col_hbm, lap_hbm, er, ec, degp, stage, dinv, acc,
                 deg_sh):
    cid = lax.axis_index("c")
    sid = lax.axis_index("s")
    epw = _E // _NS
    lane = lax.iota(jnp.int32, _L)
    zv = jnp.zeros((_L,), jnp.float32)
    ones = jnp.ones((_L,), jnp.float32)

    pltpu.sync_copy(row_hbm.at[pl.ds(sid * epw, epw)], er.at[pl.ds(0, epw)])
    pltpu.sync_copy(col_hbm.at[pl.ds(sid * epw, epw)], ec.at[pl.ds(0, epw)])

    def _z1(i, _):
        degp[pl.ds(i * _L, _L)] = zv
        return 0
    lax.fori_loop(0, _N // _L, _z1, 0)

    def _p1(i, _):
        r = er[pl.ds(i * _L, _L)]
        c = ec[pl.ds(i * _L, _L)]
        valid = r != c
        for kk in range(_L):
            plsc.addupdate_scatter(degp, [r], ones, mask=valid & (lane == kk))
        return 0
    lax.fori_loop(0, epw // _L, _p1, 0)

    pltpu.sync_copy(degp, deg_sh.at[sid])
    plsc.subcore_barrier()
    pltpu.sync_copy(deg_sh, stage)

    def _m1(j, _):
        s = stage[0, pl.ds(j * _L, _L)]
        for rr in range(1, _NS):
            s = s + stage[rr, pl.ds(j * _L, _L)]
        dinv[pl.ds(j * _L, _L)] = jnp.where(s > 0, _fast_rsqrt(s), 0.0)
        return 0
    lax.fori_loop(0, _N // _L, _m1, 0)

    pltpu.sync_copy(row_hbm, er)
    pltpu.sync_copy(col_hbm, ec)

    def _z2(rr, _):
        def _z2i(j, _2):
            acc[rr, pl.ds(j * _L, _L)] = zv
            return 0
        lax.fori_loop(0, _N // _L, _z2i, 0)
        return 0
    lax.fori_loop(0, _RPS, _z2, 0)

    base = (cid * _NS + sid) * _RPS

    def _p2(i, _):
        r = er[pl.ds(i * _L, _L)]
        c = ec[pl.ds(i * _L, _L)]
        dr = plsc.load_gather(dinv, [r])
        dc = plsc.load_gather(dinv, [c])
        w = -(dr * dc)
        cb = c - base
        inwin = (cb >= 0) & (cb < _RPS) & (r != c)
        for kk in range(_L):
            plsc.addupdate_scatter(acc, [cb, r], w, mask=inwin & (lane == kk))
        return 0
    lax.fori_loop(0, _E // _L, _p2, 0)

    pltpu.sync_copy(acc, lap_hbm.at[pl.ds(base, _RPS)])


def _build_lap(row, col):
    mesh = plsc.VectorSubcoreMesh(core_axis_name="c", subcore_axis_name="s")
    f = functools.partial(
        pl.kernel,
        mesh=mesh,
        out_type=jax.ShapeDtypeStruct((_N, _N), jnp.float32),
        compiler_params=pltpu.CompilerParams(needs_layout_passes=False),
        scratch_types=[
            pltpu.VMEM((_E,), jnp.int32),
            pltpu.VMEM((_E,), jnp.int32),
            pltpu.VMEM((_N,), jnp.float32),
            pltpu.VMEM((_NS, _N), jnp.float32),
            pltpu.VMEM((_N,), jnp.float32),
            pltpu.VMEM((_RPS, _N), jnp.float32),
            pltpu.VMEM_SHARED((_NS, _N), jnp.float32),
        ],
    )(_lap_sc_body)
    return f(row, col)


def _att_lr_body(x_ref, u1_ref, uw3_ref, u2_ref, be_ref, ve_ref, w1_ref,
                 w2_ref, lhs_ref, rhs_ref, vm_s):
    xb = x_ref[0]
    vflat = lax.dot_general(u1_ref[...], xb, (((1,), (0,)), ((), ())),
                            preferred_element_type=jnp.float32)
    for t in range(_T):
        vm_s[t:t + 1, :] = vflat[:, t * _F:(t + 1) * _F]
    rq = lax.dot_general(xb, uw3_ref[...], (((1,), (0,)), ((), ())),
                         preferred_element_type=jnp.float32)
    lhs_s = lax.dot_general(vm_s[...].astype(jnp.bfloat16), u2_ref[...],
                            (((1,), (0,)), ((), ())),
                            preferred_element_type=jnp.float32)
    product = lax.dot_general(lhs_s, rq[:, 0:_T], (((1,), (0,)), ((), ())),
                              preferred_element_type=jnp.float32)
    e0 = jax.nn.sigmoid(product + be_ref[...])
    emat = jnp.dot(ve_ref[...], e0, preferred_element_type=jnp.float32)
    m = jnp.max(emat, axis=0, keepdims=True)
    ex = jnp.exp(emat - m)
    xt = ex / jnp.sum(ex, axis=0, keepdims=True)

    a = jnp.dot(xt, w1_ref[...], preferred_element_type=jnp.float32)
    s1 = x_ref[0, :, 0:_F].astype(jnp.float32) * a[0:1, 0:1]
    for t in range(1, _T):
        s1 += x_ref[0, :, t * _F:(t + 1) * _F].astype(jnp.float32) * a[t:t + 1, 0:1]
    lhs_ref[0] = lax.dot_general(s1, w2_ref[...], (((1,), (0,)), ((), ())),
                                 preferred_element_type=jnp.float32)
    rhs_ref[0] = lax.dot_general(xt, rq[:, _T:2 * _T], (((0,), (1,)), ((), ())),
                                 preferred_element_type=jnp.float32)


def _att_lr(x2, u1r, uw3blk, u2, be2, ve, w1c, w2):
    return pl.pallas_call(
        _att_lr_body,
        grid=(_B,),
        in_specs=[
            pl.BlockSpec((1, _N, _T * _F), lambda b: (b, 0, 0)),
            pl.BlockSpec((1, _N), lambda b: (0, 0)),
            pl.BlockSpec((_T * _F, 2 * _T), lambda b: (0, 0)),
            pl.BlockSpec((_F, _N), lambda b: (0, 0)),
            pl.BlockSpec((_T, _T), lambda b: (0, 0)),
            pl.BlockSpec((_T, _T), lambda b: (0, 0)),
            pl.BlockSpec((_T, 1), lambda b: (0, 0)),
            pl.BlockSpec((_F, _T), lambda b: (0, 0)),
        ],
        out_specs=[
            pl.BlockSpec((1, _N, _T), lambda b: (b, 0, 0)),
            pl.BlockSpec((1, _T, _N), lambda b: (b, 0, 0)),
        ],
        out_shape=[
            jax.ShapeDtypeStruct((_B, _N, _T), jnp.float32),
            jax.ShapeDtypeStruct((_B, _T, _N), jnp.float32),
        ],
        scratch_shapes=[pltpu.VMEM((_T, _F), jnp.float32)],
    )(x2, u1r, uw3blk, u2, be2, ve, w1c, w2)


def _satt_body(rhs_ref, lhs_ref, bst_ref, vst_ref, out_ref):
    pt = lax.dot_general(rhs_ref[0], lhs_ref[0], (((0,), (1,)), ((), ())),
                         preferred_element_type=jnp.float32)
    sigt = jax.nn.sigmoid(pt + bst_ref[...].astype(jnp.float32)).astype(jnp.bfloat16)
    st = jnp.dot(sigt, vst_ref[...], preferred_element_type=jnp.float32)
    m = jnp.max(st, axis=1, keepdims=True)
    ex = jnp.exp(st - m)
    out_ref[0] = (ex / jnp.sum(ex, axis=1, keepdims=True)).astype(jnp.bfloat16)


def _satt(lhs, rhs, bsT, vsT):
    jb = 512
    nj = _N // jb
    return pl.pallas_call(
        _satt_body,
        grid=(_B, nj),
        in_specs=[
            pl.BlockSpec((1, _T, jb), lambda b, j: (b, 0, j)),
            pl.BlockSpec((1, _N, _T), lambda b, j: (b, 0, 0)),
            pl.BlockSpec((jb, _N), lambda b, j: (j, 0)),
            pl.BlockSpec((_N, _N), lambda b, j: (0, 0)),
        ],
        out_specs=pl.BlockSpec((1, jb, _N), lambda b, j: (b, j, 0)),
        out_shape=jax.ShapeDtypeStruct((_B, _N, _N), jnp.bfloat16),
    )(rhs, lhs, bsT, vsT)


def _cheb_final_body(satt_ref, lap_ref, x_ref, w0_ref, w1_ref, w2_ref, b_ref,
                     twd_ref, rwt_ref, tb_ref, rb_ref, gam_ref, bet_ref,
                     out_ref, t0_s, t1_s, g_s):
    sT = satt_ref[0]
    xb = x_ref[0]
    t0_s[...] = lax.dot_general(sT, xb, (((0,), (0,)), ((), ())),
                                preferred_element_type=jnp.float32)
    m = lap_ref[...] * sT
    t1_s[...] = jnp.dot(m, xb, preferred_element_type=jnp.float32)
    t2 = 2.0 * jnp.dot(m, t1_s[...].astype(jnp.bfloat16),
                       preferred_element_type=jnp.float32) - t0_s[...]
    for t in range(_T):
        sl = slice(t * _F, (t + 1) * _F)
        acc = jnp.dot(t0_s[:, sl], w0_ref[...], preferred_element_type=jnp.float32)
        acc += jnp.dot(t1_s[:, sl], w1_ref[...], preferred_element_type=jnp.float32)
        acc += jnp.dot(t2[:, sl], w2_ref[...], preferred_element_type=jnp.float32)
        acc += b_ref[...]
        g_s[:, t * _CC:(t + 1) * _CC] = jnp.maximum(acc, 0.0).astype(jnp.bfloat16)
    for t in range(_T):
        acc = jnp.dot(x_ref[0, :, t * _F:(t + 1) * _F], rwt_ref[...],
                      preferred_element_type=jnp.float32) + rb_ref[...]
        for d in range(3):
            tp = t + d - 1
            if 0 <= tp < _T:
                acc += jnp.dot(g_s[:, tp * _CC:(tp + 1) * _CC], twd_ref[d],
                               preferred_element_type=jnp.float32)
        acc += tb_ref[...]
        z = jnp.maximum(acc, 0.0)
        mu = jnp.mean(z, axis=1, keepdims=True)
        dz = z - mu
        var = jnp.mean(dz * dz, axis=1, keepdims=True)
        out_ref[0, :, t * _CT:(t + 1) * _CT] = (
            dz * lax.rsqrt(var + 1e-5) * gam_ref[...] + bet_ref[...])


def _cheb_final(sattT, lap, x2, w0, w1, w2, bc, twd, rwt, tb1, rb1, gam1, bet1):
    return pl.pallas_call(
        _cheb_final_body,
        grid=(_B,),
        in_specs=[
            pl.BlockSpec((1, _N, _N), lambda b: (b, 0, 0)),
            pl.BlockSpec((_N, _N), lambda b: (0, 0)),
            pl.BlockSpec((1, _N, _T * _F), lambda b: (b, 0, 0)),
            pl.BlockSpec((_F, _CC), lambda b: (0, 0)),
            pl.BlockSpec((_F, _CC), lambda b: (0, 0)),
            pl.BlockSpec((_F, _CC), lambda b: (0, 0)),
            pl.BlockSpec((1, _CC), lambda b: (0, 0)),
            pl.BlockSpec((3, _CC, _CT), lambda b: (0, 0, 0)),
            pl.BlockSpec((_F, _CT), lambda b: (0, 0)),
            pl.BlockSpec((1, _CT), lambda b: (0, 0)),
            pl.BlockSpec((1, _CT), lambda b: (0, 0)),
            pl.BlockSpec((1, _CT), lambda b: (0, 0)),
            pl.BlockSpec((1, _CT), lambda b: (0, 0)),
        ],
        out_specs=pl.BlockSpec((1, _N, _T * _CT), lambda b: (b, 0, 0)),
        out_shape=jax.ShapeDtypeStruct((_B, _N, _T * _CT), jnp.float32),
        scratch_shapes=[pltpu.VMEM((_N, _T * _F), jnp.float32),
                        pltpu.VMEM((_N, _T * _F), jnp.float32),
                        pltpu.VMEM((_N, _T * _CC), jnp.bfloat16)],
    )(sattT, lap, x2, w0, w1, w2, bc, twd, rwt, tb1, rb1, gam1, bet1)


@jax.jit
def _run(x, edge_index, U1, U2, U3, be, Ve, W1, W2, W3, bs, Vs, cheb_w,
         cheb_b, tw, tb, rw, rb, gamma, beta):
    b, n, f, t = x.shape
    bf = jnp.bfloat16
    x2 = jnp.transpose(x.astype(bf), (0, 1, 3, 2)).reshape(b, n, t * f)

    lap = _build_lap(edge_index[0], edge_index[1])

    eyet = jnp.eye(t, dtype=jnp.float32)
    uw3blk = jnp.concatenate(
        [jnp.kron(eyet, U3.reshape(f, 1)), jnp.kron(eyet, W3.reshape(f, 1))],
        axis=1).astype(bf)
    lhs, rhs = _att_lr(x2, U1.reshape(1, n).astype(bf), uw3blk, U2.astype(bf),
                       be[0], Ve, W1.reshape(t, 1), W2)
    sattT = _satt(lhs, rhs, jnp.transpose(bs[0].astype(bf)),
                  jnp.transpose(Vs.astype(bf)))

    twd = jnp.transpose(tw[:, :, 0, :], (2, 1, 0)).astype(bf)
    rwt = jnp.transpose(rw[:, :, 0, 0]).astype(bf)
    zout = _cheb_final(sattT, lap.astype(bf), x2, cheb_w[0], cheb_w[1],
                       cheb_w[2], cheb_b.reshape(1, _CC), twd, rwt,
                       tb.reshape(1, _CT), rb.reshape(1, _CT),
                       gamma.reshape(1, _CT), beta.reshape(1, _CT))
    return jnp.transpose(zout.reshape(b, n, _T, _CT), (0, 1, 3, 2))


def kernel(x, edge_index, U1, U2, U3, be, Ve, W1, W2, W3, bs, Vs, cheb_w,
           cheb_b, tw, tb, rw, rb, gamma, beta):
    return _run(x, edge_index, U1, U2, U3, be, Ve, W1, W2, W3, bs, Vs,
                cheb_w, cheb_b, tw, tb, rw, rb, gamma, beta)


# speedup vs baseline: 1.7563x; 1.1427x over previous
import functools
import jax
import jax.numpy as jnp
from jax import lax
from jax.experimental import pallas as pl
from jax.experimental.pallas import tpu as pltpu
from jax.experimental.pallas import tpu_sc as plsc

_B, _N, _F, _T = 8, 1024, 64, 12
_CC, _CT = 64, 64
_E = 16384
_NC, _NS, _L = 2, 16, 16
_RPS = _N // (_NC * _NS)


def _fast_rsqrt(s):
    y = plsc.bitcast(jnp.int32(0x5F3759DF) - (plsc.bitcast(s, jnp.int32) >> 1),
                     jnp.float32)
    for _ in range(3):
        y = y * (1.5 - 0.5 * s * y * y)
    return y


def _lap_sc_body(row_hbm, col_hbm, lap_hbm, er, ec, degp, stage, dinv, acc,
                 deg_sh):
    cid = lax.axis_index("c")
    sid = lax.axis_index("s")
    epw = _E // _NS
    lane = lax.iota(jnp.int32, _L)
    zv = jnp.zeros((_L,), jnp.float32)
    ones = jnp.ones((_L,), jnp.float32)

    pltpu.sync_copy(row_hbm.at[pl.ds(sid * epw, epw)], er.at[pl.ds(0, epw)])
    pltpu.sync_copy(col_hbm.at[pl.ds(sid * epw, epw)], ec.at[pl.ds(0, epw)])

    def _z1(i, _):
        degp[pl.ds(i * _L, _L)] = zv
        return 0
    lax.fori_loop(0, _N // _L, _z1, 0)

    def _p1(i, _):
        r = er[pl.ds(i * _L, _L)]
        c = ec[pl.ds(i * _L, _L)]
        valid = r != c
        for kk in range(_L):
            plsc.addupdate_scatter(degp, [r], ones, mask=valid & (lane == kk))
        return 0
    lax.fori_loop(0, epw // _L, _p1, 0)

    pltpu.sync_copy(degp, deg_sh.at[sid])
    plsc.subcore_barrier()
    pltpu.sync_copy(deg_sh, stage)

    def _m1(j, _):
        s = stage[0, pl.ds(j * _L, _L)]
        for rr in range(1, _NS):
            s = s + stage[rr, pl.ds(j * _L, _L)]
        dinv[pl.ds(j * _L, _L)] = jnp.where(s > 0, _fast_rsqrt(s), 0.0)
        return 0
    lax.fori_loop(0, _N // _L, _m1, 0)

    pltpu.sync_copy(row_hbm, er)
    pltpu.sync_copy(col_hbm, ec)

    def _z2(rr, _):
        def _z2i(j, _2):
            acc[rr, pl.ds(j * _L, _L)] = zv
            return 0
        lax.fori_loop(0, _N // _L, _z2i, 0)
        return 0
    lax.fori_loop(0, _RPS, _z2, 0)

    base = (cid * _NS + sid) * _RPS

    def _p2(i, _):
        r = er[pl.ds(i * _L, _L)]
        c = ec[pl.ds(i * _L, _L)]
        dr = plsc.load_gather(dinv, [r])
        dc = plsc.load_gather(dinv, [c])
        w = -(dr * dc)
        cb = c - base
        inwin = (cb >= 0) & (cb < _RPS) & (r != c)
        for kk in range(_L):
            plsc.addupdate_scatter(acc, [cb, r], w, mask=inwin & (lane == kk))
        return 0
    lax.fori_loop(0, _E // _L, _p2, 0)

    pltpu.sync_copy(acc, lap_hbm.at[pl.ds(base, _RPS)])


def _build_lap(row, col):
    mesh = plsc.VectorSubcoreMesh(core_axis_name="c", subcore_axis_name="s")
    f = functools.partial(
        pl.kernel,
        mesh=mesh,
        out_type=jax.ShapeDtypeStruct((_N, _N), jnp.float32),
        compiler_params=pltpu.CompilerParams(needs_layout_passes=False),
        scratch_types=[
            pltpu.VMEM((_E,), jnp.int32),
            pltpu.VMEM((_E,), jnp.int32),
            pltpu.VMEM((_N,), jnp.float32),
            pltpu.VMEM((_NS, _N), jnp.float32),
            pltpu.VMEM((_N,), jnp.float32),
            pltpu.VMEM((_RPS, _N), jnp.float32),
            pltpu.VMEM_SHARED((_NS, _N), jnp.float32),
        ],
    )(_lap_sc_body)
    return f(row, col)


def _att_lr_body(x_ref, u1_ref, uw3_ref, u2_ref, be_ref, ve_ref, w1_ref,
                 w2_ref, lhs_ref, rhs_ref, vm_s):
    xb = x_ref[0]
    vflat = lax.dot_general(u1_ref[...], xb, (((1,), (0,)), ((), ())),
                            preferred_element_type=jnp.float32)
    for t in range(_T):
        vm_s[t:t + 1, :] = vflat[:, t * _F:(t + 1) * _F]
    rq = lax.dot_general(xb, uw3_ref[...], (((1,), (0,)), ((), ())),
                         preferred_element_type=jnp.float32)
    lhs_s = lax.dot_general(vm_s[...].astype(jnp.bfloat16), u2_ref[...],
                            (((1,), (0,)), ((), ())),
                            preferred_element_type=jnp.float32)
    product = lax.dot_general(lhs_s, rq[:, 0:_T], (((1,), (0,)), ((), ())),
                              preferred_element_type=jnp.float32)
    e0 = jax.nn.sigmoid(product + be_ref[...])
    emat = jnp.dot(ve_ref[...], e0, preferred_element_type=jnp.float32)
    m = jnp.max(emat, axis=0, keepdims=True)
    ex = jnp.exp(emat - m)
    xt = ex / jnp.sum(ex, axis=0, keepdims=True)

    a = jnp.dot(xt, w1_ref[...], preferred_element_type=jnp.float32)
    s1 = x_ref[0, :, 0:_F].astype(jnp.float32) * a[0:1, 0:1]
    for t in range(1, _T):
        s1 += x_ref[0, :, t * _F:(t + 1) * _F].astype(jnp.float32) * a[t:t + 1, 0:1]
    lhs_ref[0] = lax.dot_general(s1, w2_ref[...], (((1,), (0,)), ((), ())),
                                 preferred_element_type=jnp.float32)
    rhs_ref[0] = lax.dot_general(xt, rq[:, _T:2 * _T], (((0,), (1,)), ((), ())),
                                 preferred_element_type=jnp.float32)


def _att_lr(x2, u1r, uw3blk, u2, be2, ve, w1c, w2):
    return pl.pallas_call(
        _att_lr_body,
        grid=(_B,),
        in_specs=[
            pl.BlockSpec((1, _N, _T * _F), lambda b: (b, 0, 0)),
            pl.BlockSpec((1, _N), lambda b: (0, 0)),
            pl.BlockSpec((_T * _F, 2 * _T), lambda b: (0, 0)),
            pl.BlockSpec((_F, _N), lambda b: (0, 0)),
            pl.BlockSpec((_T, _T), lambda b: (0, 0)),
            pl.BlockSpec((_T, _T), lambda b: (0, 0)),
            pl.BlockSpec((_T, 1), lambda b: (0, 0)),
            pl.BlockSpec((_F, _T), lambda b: (0, 0)),
        ],
        out_specs=[
            pl.BlockSpec((1, _N, _T), lambda b: (b, 0, 0)),
            pl.BlockSpec((1, _T, _N), lambda b: (b, 0, 0)),
        ],
        out_shape=[
            jax.ShapeDtypeStruct((_B, _N, _T), jnp.float32),
            jax.ShapeDtypeStruct((_B, _T, _N), jnp.float32),
        ],
        scratch_shapes=[pltpu.VMEM((_T, _F), jnp.float32)],
    )(x2, u1r, uw3blk, u2, be2, ve, w1c, w2)


def _fused_body(rhs_ref, lhs_ref, bst_ref, vst_ref, lap_ref, x_ref,
                w0_ref, w1_ref, w2_ref, b_ref, twd_ref, rwt_ref, tb_ref,
                rb_ref, gam_ref, bet_ref, out_ref, satt_s, t0_s, t1_s, g_s):
    jb = 512
    for j in range(_N // jb):
        sl = slice(j * jb, (j + 1) * jb)
        pt = lax.dot_general(rhs_ref[0, :, sl], lhs_ref[0],
                             (((0,), (1,)), ((), ())),
                             preferred_element_type=jnp.float32)
        sigt = jax.nn.sigmoid(
            pt + bst_ref[sl, :].astype(jnp.float32)).astype(jnp.bfloat16)
        st = jnp.dot(sigt, vst_ref[...], preferred_element_type=jnp.float32)
        mx = jnp.max(st, axis=1, keepdims=True)
        ex = jnp.exp(st - mx)
        satt_s[sl, :] = (ex / jnp.sum(ex, axis=1, keepdims=True)).astype(jnp.bfloat16)

    sT = satt_s[...]
    xb = x_ref[0]
    t0_s[...] = lax.dot_general(sT, xb, (((0,), (0,)), ((), ())),
                                preferred_element_type=jnp.float32)
    m = lap_ref[...] * sT
    t1_s[...] = jnp.dot(m, xb, preferred_element_type=jnp.float32)
    t2 = 2.0 * jnp.dot(m, t1_s[...].astype(jnp.bfloat16),
                       preferred_element_type=jnp.float32) - t0_s[...]
    for t in range(_T):
        sl = slice(t * _F, (t + 1) * _F)
        acc = jnp.dot(t0_s[:, sl], w0_ref[...], preferred_element_type=jnp.float32)
        acc += jnp.dot(t1_s[:, sl], w1_ref[...], preferred_element_type=jnp.float32)
        acc += jnp.dot(t2[:, sl], w2_ref[...], preferred_element_type=jnp.float32)
        acc += b_ref[...]
        g_s[:, t * _CC:(t + 1) * _CC] = jnp.maximum(acc, 0.0).astype(jnp.bfloat16)
    for t in range(_T):
        acc = jnp.dot(x_ref[0, :, t * _F:(t + 1) * _F], rwt_ref[...],
                      preferred_element_type=jnp.float32) + rb_ref[...]
        for d in range(3):
            tp = t + d - 1
            if 0 <= tp < _T:
                acc += jnp.dot(g_s[:, tp * _CC:(tp + 1) * _CC], twd_ref[d],
                               preferred_element_type=jnp.float32)
        acc += tb_ref[...]
        z = jnp.maximum(acc, 0.0)
        mu = jnp.mean(z, axis=1, keepdims=True)
        dz = z - mu
        var = jnp.mean(dz * dz, axis=1, keepdims=True)
        out_ref[0, t] = dz * lax.rsqrt(var + 1e-5) * gam_ref[...] + bet_ref[...]


def _fused(lhs, rhs, bsT, vsT, lap, x2, w0, w1, w2, bc, twd, rwt, tb1, rb1,
           gam1, bet1):
    return pl.pallas_call(
        _fused_body,
        grid=(_B,),
        in_specs=[
            pl.BlockSpec((1, _T, _N), lambda b: (b, 0, 0)),
            pl.BlockSpec((1, _N, _T), lambda b: (b, 0, 0)),
            pl.BlockSpec((_N, _N), lambda b: (0, 0)),
            pl.BlockSpec((_N, _N), lambda b: (0, 0)),
            pl.BlockSpec((_N, _N), lambda b: (0, 0)),
            pl.BlockSpec((1, _N, _T * _F), lambda b: (b, 0, 0)),
            pl.BlockSpec((_F, _CC), lambda b: (0, 0)),
            pl.BlockSpec((_F, _CC), lambda b: (0, 0)),
            pl.BlockSpec((_F, _CC), lambda b: (0, 0)),
            pl.BlockSpec((1, _CC), lambda b: (0, 0)),
            pl.BlockSpec((3, _CC, _CT), lambda b: (0, 0, 0)),
            pl.BlockSpec((_F, _CT), lambda b: (0, 0)),
            pl.BlockSpec((1, _CT), lambda b: (0, 0)),
            pl.BlockSpec((1, _CT), lambda b: (0, 0)),
            pl.BlockSpec((1, _CT), lambda b: (0, 0)),
            pl.BlockSpec((1, _CT), lambda b: (0, 0)),
        ],
        out_specs=pl.BlockSpec((1, _T, _N, _CT), lambda b: (b, 0, 0, 0)),
        out_shape=jax.ShapeDtypeStruct((_B, _T, _N, _CT), jnp.float32),
        scratch_shapes=[pltpu.VMEM((_N, _N), jnp.bfloat16),
                        pltpu.VMEM((_N, _T * _F), jnp.float32),
                        pltpu.VMEM((_N, _T * _F), jnp.float32),
                        pltpu.VMEM((_N, _T * _CC), jnp.bfloat16)],
    )(rhs, lhs, bsT, vsT, lap, x2, w0, w1, w2, bc, twd, rwt, tb1, rb1,
      gam1, bet1)


@jax.jit
def _run(x, edge_index, U1, U2, U3, be, Ve, W1, W2, W3, bs, Vs, cheb_w,
         cheb_b, tw, tb, rw, rb, gamma, beta):
    b, n, f, t = x.shape
    bf = jnp.bfloat16
    x2 = jnp.transpose(x.astype(bf), (0, 1, 3, 2)).reshape(b, n, t * f)

    lap = _build_lap(edge_index[0], edge_index[1])

    eyet = jnp.eye(t, dtype=jnp.float32)
    uw3blk = jnp.concatenate(
        [jnp.kron(eyet, U3.reshape(f, 1)), jnp.kron(eyet, W3.reshape(f, 1))],
        axis=1).astype(bf)
    lhs, rhs = _att_lr(x2, U1.reshape(1, n).astype(bf), uw3blk, U2.astype(bf),
                       be[0], Ve, W1.reshape(t, 1), W2)

    twd = jnp.transpose(tw[:, :, 0, :], (2, 1, 0)).astype(bf)
    rwt = jnp.transpose(rw[:, :, 0, 0]).astype(bf)
    zout = _fused(lhs, rhs, jnp.transpose(bs[0].astype(bf)),
                  jnp.transpose(Vs.astype(bf)), lap.astype(bf), x2,
                  cheb_w[0], cheb_w[1], cheb_w[2], cheb_b.reshape(1, _CC),
                  twd, rwt, tb.reshape(1, _CT), rb.reshape(1, _CT),
                  gamma.reshape(1, _CT), beta.reshape(1, _CT))
    return jnp.transpose(zout, (0, 2, 3, 1))


def kernel(x, edge_index, U1, U2, U3, be, Ve, W1, W2, W3, bs, Vs, cheb_w,
           cheb_b, tw, tb, rw, rb, gamma, beta):
    return _run(x, edge_index, U1, U2, U3, be, Ve, W1, W2, W3, bs, Vs,
                cheb_w, cheb_b, tw, tb, rw, rb, gamma, beta)
